# Initial kernel scaffold; baseline (speedup 1.0000x reference)
import jax
import jax.numpy as jnp
from jax.experimental import pallas as pl


def kernel(x, trans_row, trans_col, trans_val, spiral_indices, W, b):
    raise NotImplementedError("write your pallas kernel here")


# speedup vs baseline: 1.8817x; 1.8817x over previous
import functools

import jax
import jax.numpy as jnp
from jax import lax
from jax.experimental import pallas as pl
from jax.experimental.pallas import tpu as pltpu
from jax.experimental.pallas import tpu_sc as plsc

N_IN = 25000
N_OUT = 50000
NNZ = 200000
C_IN = 128
C_OUT = 32
L = 9

CC = 32
NQ = 4
BATCH = 128
N_PAD = 50176
NNZ_PAD = 200704
NB_A = 98
ROWS_PER_TILE = N_PAD // 16
ZROWS = 224
N_PAD_C = 50048
NB_C = N_PAD_C // BATCH
MM_BLK = 512

_SPLAT_DNUMS = lax.GatherDimensionNumbers(
    offset_dims=(), collapsed_slice_dims=(0,), start_index_map=(0,))


def _lane_splat(vec16, lane):
    idx = jnp.full((16, 1), lane, jnp.int32)
    return lax.gather(vec16, idx, dimension_numbers=_SPLAT_DNUMS,
                      slice_sizes=(1,),
                      mode=lax.GatherScatterMode.PROMISE_IN_BOUNDS)


def _pool_body(x0, x1, x2, x3, rowp, colp, valp,
               out0, out1, out2, out3,
               spmem, rows, colbuf, rowbuf, valbuf, zbuf, sem):
    c = lax.axis_index("c")
    t = lax.axis_index("s")
    xs = (x0, x1, x2, x3)
    outs = (out0, out1, out2, out3)

    zero16 = jnp.zeros((16,), jnp.float32)

    def _zb(i, carry):
        zbuf[i, pl.ds(0, 16)] = zero16
        zbuf[i, pl.ds(16, 16)] = zero16
        return carry

    lax.fori_loop(0, ZROWS, _zb, 0)

    for p in range(2):
        for z in range(ROWS_PER_TILE // ZROWS):
            pltpu.sync_copy(
                zbuf, spmem.at[pl.ds(t * ROWS_PER_TILE + z * ZROWS, ZROWS)])
        plsc.subcore_barrier()

        for h in range(2):
            q = 2 * h + p
            xq = xs[q]

            @pl.when(c == h)
            def _scatter(xq=xq):
                def _batch(bidx, carry):
                    base = (t * NB_A + bidx) * BATCH
                    pltpu.sync_copy(colp.at[pl.ds(base, BATCH)], colbuf)
                    pltpu.async_copy(xq.at[colbuf], rows, sem).wait()
                    pltpu.sync_copy(valp.at[pl.ds(base, BATCH)], valbuf)
                    pltpu.sync_copy(rowp.at[pl.ds(base, BATCH)], rowbuf)

                    def _group(g, gcarry):
                        vv = valbuf[pl.ds(g * 16, 16)]

                        def _row(i16, rcarry):
                            v = _lane_splat(vv, i16)
                            i = g * 16 + i16
                            for k in range(2):
                                sl = pl.ds(16 * k, 16)
                                rows[i, sl] = rows[i, sl] * v
                            return rcarry

                        lax.fori_loop(0, 16, _row, 0)
                        return gcarry

                    lax.fori_loop(0, 8, _group, 0)
                    pltpu.sync_copy(rows, spmem.at[rowbuf], add=True)
                    return carry

                lax.fori_loop(0, NB_A, _batch, 0)

        plsc.subcore_barrier()

        for h in range(2):
            q = 2 * h + p
            outq = outs[q]

            @pl.when(c == h)
            def _copy_out(outq=outq):
                sl = pl.ds(t * ROWS_PER_TILE, ROWS_PER_TILE)
                pltpu.sync_copy(spmem.at[sl], outq.at[sl])

        plsc.subcore_barrier()


def _matmul_body(x0, x1, x2, x3, w, *outs):
    cat = jnp.concatenate([x0[...], x1[...], x2[...], x3[...]], axis=1)
    acc = lax.dot_general(cat, w[...], (((1,), (0,)), ((), ())),
                          preferred_element_type=jnp.float32)
    for j in range(L):
        outs[j][...] = acc[:, C_OUT * j:C_OUT * (j + 1)]


def _spiral_body(st, y0, y1, y2, y3, y4, y5, y6, y7, y8, bias, out,
                 i0, i1, i2, i3, i4, i5, i6, i7, i8,
                 g0, g1, g2, g3, g4, g5, g6, g7, g8,
                 obuf, bvm, sem):
    c = lax.axis_index("c")
    s = lax.axis_index("s")
    wid = s * 2 + c
    ys = (y0, y1, y2, y3, y4, y5, y6, y7, y8)
    idxs = (i0, i1, i2, i3, i4, i5, i6, i7, i8)
    gs = (g0, g1, g2, g3, g4, g5, g6, g7, g8)

    pltpu.sync_copy(bias, bvm)
    b0 = bvm[pl.ds(0, 16)]
    b1 = bvm[pl.ds(16, 16)]

    count = (NB_C - wid + 31) // 32

    def _node_batch(n, carry):
        base = (wid + n * 32) * BATCH
        for j in range(L):
            pltpu.sync_copy(st.at[j, pl.ds(base, BATCH)], idxs[j])
        descs = [pltpu.async_copy(ys[j].at[idxs[j]], gs[j], sem)
                 for j in range(L)]
        for d in descs:
            d.wait()

        def _node(i, ncarry):
            for k in range(2):
                sl = pl.ds(16 * k, 16)
                acc = gs[0][i, sl]
                for j in range(1, L):
                    acc = acc + gs[j][i, sl]
                acc = acc + (b0 if k == 0 else b1)
                e = jnp.exp(acc) - 1.0
                obuf[i, sl] = jnp.where(acc > 0.0, acc, e)
            return ncarry

        lax.fori_loop(0, BATCH, _node, 0)
        pltpu.sync_copy(obuf, out.at[pl.ds(base, BATCH)])
        return carry

    lax.fori_loop(0, count, _node_batch, 0)


def _pool_call(x_chunks, rowp, colp, valp):
    mesh = plsc.VectorSubcoreMesh(core_axis_name="c", subcore_axis_name="s")
    f = pl.kernel(
        _pool_body,
        out_type=[jax.ShapeDtypeStruct((N_PAD, CC), jnp.float32)
                  for _ in range(NQ)],
        mesh=mesh,
        scratch_types=[
            pltpu.VMEM_SHARED((N_PAD, CC), jnp.float32),
            pltpu.VMEM((BATCH, CC), jnp.float32),
            pltpu.VMEM((BATCH,), jnp.int32),
            pltpu.VMEM((BATCH,), jnp.int32),
            pltpu.VMEM((BATCH,), jnp.float32),
            pltpu.VMEM((ZROWS, CC), jnp.float32),
            pltpu.SemaphoreType.DMA,
        ],
        compiler_params=pltpu.CompilerParams(use_tc_tiling_on_sc=False),
    )
    return f(*x_chunks, rowp, colp, valp)


def _matmul_call(pooled_chunks, w_perm):
    grid = (N_PAD // MM_BLK,)
    in_specs = ([pl.BlockSpec((MM_BLK, CC), lambda i: (i, 0))
                 for _ in range(NQ)]
                + [pl.BlockSpec((C_IN, L * C_OUT), lambda i: (0, 0))])
    out_specs = [pl.BlockSpec((MM_BLK, C_OUT), lambda i: (i, 0))
                 for _ in range(L)]
    return pl.pallas_call(
        _matmul_body,
        grid=grid,
        in_specs=in_specs,
        out_specs=out_specs,
        out_shape=[jax.ShapeDtypeStruct((N_PAD, C_OUT), jnp.float32)
                   for _ in range(L)],
    )(*pooled_chunks, w_perm)


def _spiral_call(st, ys, bias):
    mesh = plsc.VectorSubcoreMesh(core_axis_name="c", subcore_axis_name="s")
    f = pl.kernel(
        _spiral_body,
        out_type=jax.ShapeDtypeStruct((N_PAD_C, C_OUT), jnp.float32),
        mesh=mesh,
        scratch_types=(
            [pltpu.VMEM((BATCH,), jnp.int32) for _ in range(L)]
            + [pltpu.VMEM((BATCH, C_OUT), jnp.float32) for _ in range(L)]
            + [
                pltpu.VMEM((BATCH, C_OUT), jnp.float32),
                pltpu.VMEM((C_OUT,), jnp.float32),
                pltpu.SemaphoreType.DMA,
            ]
        ),
        compiler_params=pltpu.CompilerParams(use_tc_tiling_on_sc=False),
    )
    return f(st, *ys, bias)


def kernel(x, trans_row, trans_col, trans_val, spiral_indices, W, b):
    x2 = x[0]
    x_chunks = [x2[:, CC * q:CC * (q + 1)] for q in range(NQ)]

    pad = NNZ_PAD - NNZ
    rowp = jnp.pad(trans_row.astype(jnp.int32), (0, pad))
    colp = jnp.pad(trans_col.astype(jnp.int32), (0, pad))
    valp = jnp.pad(trans_val, (0, pad))

    st = jnp.pad(spiral_indices.astype(jnp.int32),
                 ((0, N_PAD_C - N_OUT), (0, 0))).T

    w_perm = W.reshape(L, C_IN, C_OUT).transpose(1, 0, 2).reshape(
        C_IN, L * C_OUT)

    pooled_chunks = _pool_call(x_chunks, rowp, colp, valp)
    ys = _matmul_call(pooled_chunks, w_perm)
    out = _spiral_call(st, ys, b)

    return out[:N_OUT].reshape(1, N_OUT, C_OUT)


# speedup vs baseline: 3.0912x; 1.6428x over previous
import functools

import jax
import jax.numpy as jnp
from jax import lax
from jax.experimental import pallas as pl
from jax.experimental.pallas import tpu as pltpu
from jax.experimental.pallas import tpu_sc as plsc

N_IN = 25000
N_OUT = 50000
NNZ = 200000
C_IN = 128
C_OUT = 32
L = 9

CC = 32
NQ = 4
BATCH = 128
N_PAD = 51200
NNZ_PAD = 200704
NB_A = 98
BPC = 7
NCHUNK_A = NB_A // BPC
ROWS_PER_TILE = N_PAD // 16
ZROWS = 200
N_PAD_C = 50048
NB_C = N_PAD_C // BATCH
MM_BLK = 2048

_SPLAT_DNUMS = lax.GatherDimensionNumbers(
    offset_dims=(), collapsed_slice_dims=(0,), start_index_map=(0,))


def _lane_splat(vec16, lane):
    idx = jnp.full((16, 1), lane, jnp.int32)
    return lax.gather(vec16, idx, dimension_numbers=_SPLAT_DNUMS,
                      slice_sizes=(1,),
                      mode=lax.GatherScatterMode.PROMISE_IN_BOUNDS)


def _pool_body(x0, x1, x2, x3, rowp, colp, valp,
               out0, out1, out2, out3,
               spmem, rows0, rows1, col7, row7, val7, zbuf, sem0, sem1):
    c = lax.axis_index("c")
    t = lax.axis_index("s")
    xs = (x0, x1, x2, x3)
    outs = (out0, out1, out2, out3)
    rows_ring = (rows0, rows1)
    sem_ring = (sem0, sem1)

    zero16 = jnp.zeros((16,), jnp.float32)

    def _zb(i, carry):
        zbuf[i, pl.ds(0, 16)] = zero16
        zbuf[i, pl.ds(16, 16)] = zero16
        return carry

    lax.fori_loop(0, ZROWS, _zb, 0)

    def _scale(rows_ref, j):
        def _grp(g, carry):
            vv = val7[j, pl.ds(g * 16, 16)]
            for i16 in range(16):
                v = _lane_splat(vv, i16)
                i = g * 16 + i16
                rows_ref[i, pl.ds(0, 16)] = rows_ref[i, pl.ds(0, 16)] * v
                rows_ref[i, pl.ds(16, 16)] = rows_ref[i, pl.ds(16, 16)] * v
            return carry

        lax.fori_loop(0, 8, _grp, 0)

    for p in range(2):
        for z in range(ROWS_PER_TILE // ZROWS):
            pltpu.sync_copy(
                zbuf, spmem.at[pl.ds(t * ROWS_PER_TILE + z * ZROWS, ZROWS)])
        plsc.subcore_barrier()

        for h in range(2):
            q = 2 * h + p
            xq = xs[q]

            @pl.when(c == h)
            def _scatter(xq=xq):
                def _chunk(ck, carry):
                    brow = t * NB_A + ck * BPC
                    pltpu.sync_copy(colp.at[pl.ds(brow, BPC)], col7)
                    pltpu.sync_copy(rowp.at[pl.ds(brow, BPC)], row7)
                    pltpu.sync_copy(valp.at[pl.ds(brow, BPC)], val7)
                    descs = {0: pltpu.async_copy(
                        xq.at[col7.at[0]], rows_ring[0], sem_ring[0])}
                    for j in range(BPC):
                        if j + 1 < BPC:
                            descs[j + 1] = pltpu.async_copy(
                                xq.at[col7.at[j + 1]],
                                rows_ring[(j + 1) % 2], sem_ring[(j + 1) % 2])
                        descs[j].wait()
                        _scale(rows_ring[j % 2], j)
                        pltpu.sync_copy(rows_ring[j % 2],
                                        spmem.at[row7.at[j]], add=True)
                    return carry

                lax.fori_loop(0, NCHUNK_A, _chunk, 0)

        plsc.subcore_barrier()

        for h in range(2):
            q = 2 * h + p
            outq = outs[q]

            @pl.when(c == h)
            def _copy_out(outq=outq):
                sl = pl.ds(t * ROWS_PER_TILE, ROWS_PER_TILE)
                pltpu.sync_copy(spmem.at[sl], outq.at[sl])

        plsc.subcore_barrier()


def _matmul_body(x0, x1, x2, x3, w, *outs):
    cat = jnp.concatenate([x0[...], x1[...], x2[...], x3[...]], axis=1)
    acc = lax.dot_general(cat, w[...], (((1,), (0,)), ((), ())),
                          preferred_element_type=jnp.float32)
    for j in range(L):
        outs[j][...] = acc[:, C_OUT * j:C_OUT * (j + 1)]


def _spiral_body(st3, y0, y1, y2, y3, y4, y5, y6, y7, y8, bias, out,
                 idxa, idxb,
                 ga0, ga1, ga2, ga3, ga4, ga5, ga6, ga7, ga8,
                 gb0, gb1, gb2, gb3, gb4, gb5, gb6, gb7, gb8,
                 obuf, bvm, sema, semb):
    c = lax.axis_index("c")
    s = lax.axis_index("s")
    wid = s * 2 + c
    ys = (y0, y1, y2, y3, y4, y5, y6, y7, y8)
    ga = (ga0, ga1, ga2, ga3, ga4, ga5, ga6, ga7, ga8)
    gb = (gb0, gb1, gb2, gb3, gb4, gb5, gb6, gb7, gb8)

    pltpu.sync_copy(bias, bvm)
    b0 = bvm[pl.ds(0, 16)]
    b1 = bvm[pl.ds(16, 16)]

    def _fire(kb, idx, g, sem):
        pltpu.sync_copy(st3.at[kb], idx)
        return [pltpu.async_copy(ys[j].at[idx.at[j]], g[j], sem)
                for j in range(L)]

    def _compute_store(kb, g):
        def _node(i, ncarry):
            for k in range(2):
                sl = pl.ds(16 * k, 16)
                acc = g[0][i, sl]
                for j in range(1, L):
                    acc = acc + g[j][i, sl]
                acc = acc + (b0 if k == 0 else b1)
                e = jnp.exp(acc) - 1.0
                obuf[i, sl] = jnp.where(acc > 0.0, acc, e)
            return ncarry

        lax.fori_loop(0, BATCH, _node, 0)
        pltpu.sync_copy(obuf, out.at[pl.ds(kb * BATCH, BATCH)])

    count = (NB_C - wid + 31) // 32
    cnt2 = count // 2

    def _pair(m, carry):
        kb_a = wid + (2 * m) * 32
        kb_b = wid + (2 * m + 1) * 32
        da = _fire(kb_a, idxa, ga, sema)
        db = _fire(kb_b, idxb, gb, semb)
        for d in da:
            d.wait()
        _compute_store(kb_a, ga)
        for d in db:
            d.wait()
        _compute_store(kb_b, gb)
        return carry

    lax.fori_loop(0, cnt2, _pair, 0)

    @pl.when(count % 2 == 1)
    def _tail():
        kb = wid + (2 * cnt2) * 32
        da = _fire(kb, idxa, ga, sema)
        for d in da:
            d.wait()
        _compute_store(kb, ga)


def _pool_call(x_chunks, rowp2, colp2, valp2):
    mesh = plsc.VectorSubcoreMesh(core_axis_name="c", subcore_axis_name="s")
    f = pl.kernel(
        _pool_body,
        out_type=[jax.ShapeDtypeStruct((N_PAD, CC), jnp.float32)
                  for _ in range(NQ)],
        mesh=mesh,
        scratch_types=[
            pltpu.VMEM_SHARED((N_PAD, CC), jnp.float32),
            pltpu.VMEM((BATCH, CC), jnp.float32),
            pltpu.VMEM((BATCH, CC), jnp.float32),
            pltpu.VMEM((BPC, BATCH), jnp.int32),
            pltpu.VMEM((BPC, BATCH), jnp.int32),
            pltpu.VMEM((BPC, BATCH), jnp.float32),
            pltpu.VMEM((ZROWS, CC), jnp.float32),
            pltpu.SemaphoreType.DMA,
            pltpu.SemaphoreType.DMA,
        ],
        compiler_params=pltpu.CompilerParams(use_tc_tiling_on_sc=False),
    )
    return f(*x_chunks, rowp2, colp2, valp2)


def _matmul_call(pooled_chunks, w_perm):
    grid = (N_PAD // MM_BLK,)
    in_specs = ([pl.BlockSpec((MM_BLK, CC), lambda i: (i, 0))
                 for _ in range(NQ)]
                + [pl.BlockSpec((C_IN, L * C_OUT), lambda i: (0, 0))])
    out_specs = [pl.BlockSpec((MM_BLK, C_OUT), lambda i: (i, 0))
                 for _ in range(L)]
    return pl.pallas_call(
        _matmul_body,
        grid=grid,
        in_specs=in_specs,
        out_specs=out_specs,
        out_shape=[jax.ShapeDtypeStruct((N_PAD, C_OUT), jnp.float32)
                   for _ in range(L)],
    )(*pooled_chunks, w_perm)


def _spiral_call(st3, ys, bias):
    mesh = plsc.VectorSubcoreMesh(core_axis_name="c", subcore_axis_name="s")
    f = pl.kernel(
        _spiral_body,
        out_type=jax.ShapeDtypeStruct((N_PAD_C, C_OUT), jnp.float32),
        mesh=mesh,
        scratch_types=(
            [pltpu.VMEM((L, BATCH), jnp.int32) for _ in range(2)]
            + [pltpu.VMEM((BATCH, C_OUT), jnp.float32) for _ in range(2 * L)]
            + [
                pltpu.VMEM((BATCH, C_OUT), jnp.float32),
                pltpu.VMEM((C_OUT,), jnp.float32),
                pltpu.SemaphoreType.DMA,
                pltpu.SemaphoreType.DMA,
            ]
        ),
        compiler_params=pltpu.CompilerParams(use_tc_tiling_on_sc=False),
    )
    return f(st3, *ys, bias)


def kernel(x, trans_row, trans_col, trans_val, spiral_indices, W, b):
    x2 = x[0]
    x_chunks = [x2[:, CC * q:CC * (q + 1)] for q in range(NQ)]

    pad = NNZ_PAD - NNZ
    rowp2 = jnp.pad(trans_row.astype(jnp.int32), (0, pad)).reshape(-1, BATCH)
    colp2 = jnp.pad(trans_col.astype(jnp.int32), (0, pad)).reshape(-1, BATCH)
    valp2 = jnp.pad(trans_val, (0, pad)).reshape(-1, BATCH)

    st3 = jnp.pad(spiral_indices.astype(jnp.int32),
                  ((0, N_PAD_C - N_OUT), (0, 0)))
    st3 = st3.T.reshape(L, NB_C, BATCH).transpose(1, 0, 2)

    w_perm = W.reshape(L, C_IN, C_OUT).transpose(1, 0, 2).reshape(
        C_IN, L * C_OUT)

    pooled_chunks = _pool_call(x_chunks, rowp2, colp2, valp2)
    ys = _matmul_call(pooled_chunks, w_perm)
    out = _spiral_call(st3, ys, b)

    return out[:N_OUT].reshape(1, N_OUT, C_OUT)


# speedup vs baseline: 5.7383x; 1.8563x over previous
import functools

import jax
import jax.numpy as jnp
from jax import lax
from jax.experimental import pallas as pl
from jax.experimental.pallas import tpu as pltpu
from jax.experimental.pallas import tpu_sc as plsc

N_IN = 25000
N_OUT = 50000
NNZ = 200000
C_IN = 128
C_OUT = 32
L = 9

CC = 32
NQ = 4
BATCH = 128
N_PAD = 51200
NNZ_PAD = 200704
NB_A = 98
BPC = 7
NCHUNK_A = NB_A // BPC
ROWS_PER_TILE = N_PAD // 16
ZROWS = 200
N_PAD_C = 50048
NB_C = N_PAD_C // BATCH
MM_BLK = 2048
NG = 3

_SPLAT_DNUMS = lax.GatherDimensionNumbers(
    offset_dims=(), collapsed_slice_dims=(0,), start_index_map=(0,))


def _lane_splat(vec16, lane):
    idx = jnp.full((16, 1), lane, jnp.int32)
    return lax.gather(vec16, idx, dimension_numbers=_SPLAT_DNUMS,
                      slice_sizes=(1,),
                      mode=lax.GatherScatterMode.PROMISE_IN_BOUNDS)


def _pool_body(xf, rowp, colp, valp, pooled,
               spmem, rows0, rows1, col7, col7t, row7, val7, zbuf,
               sem0, sem1):
    c = lax.axis_index("c")
    t = lax.axis_index("s")
    rows_ring = (rows0, rows1)
    sem_ring = (sem0, sem1)

    zero16 = jnp.zeros((16,), jnp.float32)

    def _zb(i, carry):
        zbuf[i, pl.ds(0, 16)] = zero16
        zbuf[i, pl.ds(16, 16)] = zero16
        return carry

    lax.fori_loop(0, ZROWS, _zb, 0)

    def _scale(rows_ref, j):
        def _grp(g, carry):
            vv = val7[j, pl.ds(g * 16, 16)]
            for i16 in range(16):
                v = _lane_splat(vv, i16)
                i = g * 16 + i16
                rows_ref[i, pl.ds(0, 16)] = rows_ref[i, pl.ds(0, 16)] * v
                rows_ref[i, pl.ds(16, 16)] = rows_ref[i, pl.ds(16, 16)] * v
            return carry

        lax.fori_loop(0, 8, _grp, 0)

    for p in range(2):
        for z in range(ROWS_PER_TILE // ZROWS):
            pltpu.sync_copy(
                zbuf, spmem.at[pl.ds(t * ROWS_PER_TILE + z * ZROWS, ZROWS)])
        plsc.subcore_barrier()

        for h in range(2):
            q = 2 * h + p

            @pl.when(c == h)
            def _scatter(q=q):
                def _chunk(ck, carry):
                    brow = t * NB_A + ck * BPC
                    pltpu.sync_copy(colp.at[pl.ds(brow, BPC)], col7)
                    pltpu.sync_copy(rowp.at[pl.ds(brow, BPC)], row7)
                    pltpu.sync_copy(valp.at[pl.ds(brow, BPC)], val7)
                    for j in range(BPC):
                        def _xf(g, carry2, j=j):
                            sl = pl.ds(g * 16, 16)
                            col7t[j, sl] = col7[j, sl] * 4 + q
                            return carry2

                        lax.fori_loop(0, 8, _xf, 0)
                    descs = {0: pltpu.async_copy(
                        xf.at[col7t.at[0]], rows_ring[0], sem_ring[0])}
                    for j in range(BPC):
                        if j + 1 < BPC:
                            descs[j + 1] = pltpu.async_copy(
                                xf.at[col7t.at[j + 1]],
                                rows_ring[(j + 1) % 2], sem_ring[(j + 1) % 2])
                        descs[j].wait()
                        _scale(rows_ring[j % 2], j)
                        pltpu.sync_copy(rows_ring[j % 2],
                                        spmem.at[row7.at[j]], add=True)
                    return carry

                lax.fori_loop(0, NCHUNK_A, _chunk, 0)

        plsc.subcore_barrier()

        for h in range(2):
            q = 2 * h + p

            @pl.when(c == h)
            def _copy_out(q=q):
                sl = pl.ds(t * ROWS_PER_TILE, ROWS_PER_TILE)
                pltpu.sync_copy(spmem.at[sl],
                                pooled.at[sl, pl.ds(CC * q, CC)])

        plsc.subcore_barrier()


def _matmul_body(xr, w, *outs):
    acc = lax.dot_general(xr[...], w[...], (((1,), (0,)), ((), ())),
                          preferred_element_type=jnp.float32)
    for g in range(NG):
        outs[g][...] = acc[:, C_IN * g:C_IN * (g + 1)]


def _spiral_body(st3, yf0, yf1, yf2, bias, out,
                 idxa, idxb,
                 ga0, ga1, ga2, ga3, ga4, ga5, ga6, ga7, ga8,
                 gb0, gb1, gb2, gb3, gb4, gb5, gb6, gb7, gb8,
                 obuf, bvm, sema, semb):
    c = lax.axis_index("c")
    s = lax.axis_index("s")
    wid = s * 2 + c
    yfs = (yf0, yf1, yf2)
    ga = (ga0, ga1, ga2, ga3, ga4, ga5, ga6, ga7, ga8)
    gb = (gb0, gb1, gb2, gb3, gb4, gb5, gb6, gb7, gb8)

    pltpu.sync_copy(bias, bvm)
    b0 = bvm[pl.ds(0, 16)]
    b1 = bvm[pl.ds(16, 16)]

    def _fire(kb, idx, g, sem):
        pltpu.sync_copy(st3.at[kb], idx)
        for j in range(L):
            def _xf(gg, carry, j=j):
                sl = pl.ds(gg * 16, 16)
                idx[j, sl] = idx[j, sl] * 4 + (j % 4)
                return carry

            lax.fori_loop(0, 8, _xf, 0)
        return [pltpu.async_copy(yfs[j // 4].at[idx.at[j]], g[j], sem)
                for j in range(L)]

    def _compute_store(kb, g):
        def _node(i, ncarry):
            for k in range(2):
                sl = pl.ds(16 * k, 16)
                acc = g[0][i, sl]
                for j in range(1, L):
                    acc = acc + g[j][i, sl]
                acc = acc + (b0 if k == 0 else b1)
                e = jnp.exp(acc) - 1.0
                obuf[i, sl] = jnp.where(acc > 0.0, acc, e)
            return ncarry

        lax.fori_loop(0, BATCH, _node, 0)
        pltpu.sync_copy(obuf, out.at[pl.ds(kb * BATCH, BATCH)])

    count = (NB_C - wid + 31) // 32
    cnt2 = count // 2

    def _pair(m, carry):
        kb_a = wid + (2 * m) * 32
        kb_b = wid + (2 * m + 1) * 32
        da = _fire(kb_a, idxa, ga, sema)
        db = _fire(kb_b, idxb, gb, semb)
        for d in da:
            d.wait()
        _compute_store(kb_a, ga)
        for d in db:
            d.wait()
        _compute_store(kb_b, gb)
        return carry

    lax.fori_loop(0, cnt2, _pair, 0)

    @pl.when(count % 2 == 1)
    def _tail():
        kb = wid + (2 * cnt2) * 32
        da = _fire(kb, idxa, ga, sema)
        for d in da:
            d.wait()
        _compute_store(kb, ga)


def _pool_call(xf, rowp2, colp2, valp2):
    mesh = plsc.VectorSubcoreMesh(core_axis_name="c", subcore_axis_name="s")
    f = pl.kernel(
        _pool_body,
        out_type=jax.ShapeDtypeStruct((N_PAD, C_IN), jnp.float32),
        mesh=mesh,
        scratch_types=[
            pltpu.VMEM_SHARED((N_PAD, CC), jnp.float32),
            pltpu.VMEM((BATCH, CC), jnp.float32),
            pltpu.VMEM((BATCH, CC), jnp.float32),
            pltpu.VMEM((BPC, BATCH), jnp.int32),
            pltpu.VMEM((BPC, BATCH), jnp.int32),
            pltpu.VMEM((BPC, BATCH), jnp.int32),
            pltpu.VMEM((BPC, BATCH), jnp.float32),
            pltpu.VMEM((ZROWS, CC), jnp.float32),
            pltpu.SemaphoreType.DMA,
            pltpu.SemaphoreType.DMA,
        ],
        compiler_params=pltpu.CompilerParams(use_tc_tiling_on_sc=False),
    )
    return f(xf, rowp2, colp2, valp2)


def _matmul_call(pooled, w_pad):
    grid = (N_PAD // MM_BLK,)
    in_specs = [pl.BlockSpec((MM_BLK, C_IN), lambda i: (i, 0)),
                pl.BlockSpec((C_IN, NG * C_IN), lambda i: (0, 0))]
    out_specs = [pl.BlockSpec((MM_BLK, C_IN), lambda i: (i, 0))
                 for _ in range(NG)]
    return pl.pallas_call(
        _matmul_body,
        grid=grid,
        in_specs=in_specs,
        out_specs=out_specs,
        out_shape=[jax.ShapeDtypeStruct((N_PAD, C_IN), jnp.float32)
                   for _ in range(NG)],
    )(pooled, w_pad)


def _spiral_call(st3, yfs, bias):
    mesh = plsc.VectorSubcoreMesh(core_axis_name="c", subcore_axis_name="s")
    f = pl.kernel(
        _spiral_body,
        out_type=jax.ShapeDtypeStruct((N_PAD_C, C_OUT), jnp.float32),
        mesh=mesh,
        scratch_types=(
            [pltpu.VMEM((L, BATCH), jnp.int32) for _ in range(2)]
            + [pltpu.VMEM((BATCH, C_OUT), jnp.float32) for _ in range(2 * L)]
            + [
                pltpu.VMEM((BATCH, C_OUT), jnp.float32),
                pltpu.VMEM((C_OUT,), jnp.float32),
                pltpu.SemaphoreType.DMA,
                pltpu.SemaphoreType.DMA,
            ]
        ),
        compiler_params=pltpu.CompilerParams(use_tc_tiling_on_sc=False),
    )
    return f(st3, *yfs, bias)


def kernel(x, trans_row, trans_col, trans_val, spiral_indices, W, b):
    xf = x.reshape(N_IN * NQ, CC)

    pad = NNZ_PAD - NNZ
    rowp2 = jnp.pad(trans_row.astype(jnp.int32), (0, pad)).reshape(-1, BATCH)
    colp2 = jnp.pad(trans_col.astype(jnp.int32), (0, pad)).reshape(-1, BATCH)
    valp2 = jnp.pad(trans_val, (0, pad)).reshape(-1, BATCH)

    st3 = jnp.pad(spiral_indices.astype(jnp.int32),
                  ((0, N_PAD_C - N_OUT), (0, 0)))
    st3 = st3.T.reshape(L, NB_C, BATCH).transpose(1, 0, 2)

    w_perm = W.reshape(L, C_IN, C_OUT).transpose(1, 0, 2).reshape(
        C_IN, L * C_OUT)
    w_pad = jnp.pad(w_perm, ((0, 0), (0, NG * C_IN - L * C_OUT)))

    pooled = _pool_call(xf, rowp2, colp2, valp2)
    ys = _matmul_call(pooled, w_pad)
    yfs = [y.reshape(NQ * N_PAD, CC) for y in ys]
    out = _spiral_call(st3, yfs, b)

    return out[:N_OUT].reshape(1, N_OUT, C_OUT)
